# baseline (speedup 1.0000x reference)
import jax
import jax.numpy as jnp
from jax.experimental import pallas as pl
from jax.experimental.pallas import tpu as pltpu

B, K, IN, H, OUT = 8, 256, 128, 64, 128
ALPHA = 0.2
NEG_BIG = -9000000000000000.0
LOG2E = 1.4426950408889634


def _gat_block(h, mask, W_ref, a_ref, ones, nh):
    Wh = jax.lax.dot_general(h, W_ref[...], (((1,), (0,)), ((), ())),
                             preferred_element_type=jnp.float32)
    a_s = a_ref[:, :nh] * LOG2E
    a_d = a_ref[:, nh:] * LOG2E
    s = jax.lax.dot_general(Wh, a_s, (((1,), (1,)), ((), ())),
                            preferred_element_type=jnp.float32)
    d = jax.lax.dot_general(a_d, Wh, (((1,), (1,)), ((), ())),
                            preferred_element_type=jnp.float32)
    e = s + d
    e = jnp.maximum(e, ALPHA * e)
    att = jnp.where(mask, e, NEG_BIG)
    m = jnp.max(att, axis=1, keepdims=True)
    p = jnp.exp2(att - m)
    rs = jax.lax.dot_general(p, ones, (((1,), (1,)), ((), ())),
                             preferred_element_type=jnp.float32)
    hp = jax.lax.dot_general(p, Wh, (((1,), (0,)), ((), ())),
                             preferred_element_type=jnp.float32)
    hp = hp * (1.0 / rs)
    return jnp.where(hp > 0, hp, jnp.exp(jnp.minimum(hp, 0.0)) - 1.0)


def _gat2_kernel(x_hbm, adj_hbm, W1_ref, a1_ref, W2_ref, a2_ref, out_hbm,
                 x_vm, adj_vm, out_vm, sem_x, sem_a, sem_o):
    ones = jnp.ones((1, K), dtype=jnp.float32)
    for i in range(B):
        pltpu.make_async_copy(x_hbm.at[i], x_vm.at[i], sem_x.at[i]).start()
        pltpu.make_async_copy(adj_hbm.at[i], adj_vm.at[i], sem_a.at[i]).start()
    for i in range(B):
        pltpu.make_async_copy(x_hbm.at[i], x_vm.at[i], sem_x.at[i]).wait()
        pltpu.make_async_copy(adj_hbm.at[i], adj_vm.at[i], sem_a.at[i]).wait()
        x = x_vm[i]
        mask = adj_vm[i] > 0
        h1 = _gat_block(x, mask, W1_ref, a1_ref, ones, H)
        out_vm[i] = _gat_block(h1, mask, W2_ref, a2_ref, ones, OUT)
        pltpu.make_async_copy(out_vm.at[i], out_hbm.at[i], sem_o.at[i]).start()
    for i in range(B):
        pltpu.make_async_copy(out_vm.at[i], out_hbm.at[i], sem_o.at[i]).wait()


def kernel(x, adj, W1, a1, W2, a2):
    out = pl.pallas_call(
        _gat2_kernel,
        in_specs=[
            pl.BlockSpec(memory_space=pltpu.MemorySpace.HBM),
            pl.BlockSpec(memory_space=pltpu.MemorySpace.HBM),
            pl.BlockSpec((IN, H), lambda: (0, 0)),
            pl.BlockSpec((1, 2 * H), lambda: (0, 0)),
            pl.BlockSpec((H, OUT), lambda: (0, 0)),
            pl.BlockSpec((1, 2 * OUT), lambda: (0, 0)),
        ],
        out_specs=pl.BlockSpec(memory_space=pltpu.MemorySpace.HBM),
        out_shape=jax.ShapeDtypeStruct((B, K, OUT), jnp.float32),
        scratch_shapes=[
            pltpu.VMEM((B, K, IN), jnp.float32),
            pltpu.VMEM((B, K, K), jnp.float32),
            pltpu.VMEM((B, K, OUT), jnp.float32),
            pltpu.SemaphoreType.DMA((B,)),
            pltpu.SemaphoreType.DMA((B,)),
            pltpu.SemaphoreType.DMA((B,)),
        ],
    )(x, adj, W1, a1.reshape(1, 2 * H), W2, a2.reshape(1, 2 * OUT))
    return out


# speedup vs baseline: 1.2456x; 1.2456x over previous
import jax
import jax.numpy as jnp
from jax.experimental import pallas as pl
from jax.experimental.pallas import tpu as pltpu

B, K, IN, H, OUT = 8, 256, 128, 64, 128
GRP = 4
ALPHA = 0.2
NEG_BIG = -9000000000000000.0
LOG2E = 1.4426950408889634


def _gat_block(h, mask, W_ref, a_ref, ones, nh):
    Wh = jax.lax.dot_general(h, W_ref[...], (((1,), (0,)), ((), ())),
                             preferred_element_type=jnp.float32)
    a_s = a_ref[:, :nh] * LOG2E
    a_d = a_ref[:, nh:] * LOG2E
    s = jax.lax.dot_general(Wh, a_s, (((1,), (1,)), ((), ())),
                            preferred_element_type=jnp.float32)
    d = jax.lax.dot_general(a_d, Wh, (((1,), (1,)), ((), ())),
                            preferred_element_type=jnp.float32)
    e = s + d
    e = jnp.maximum(e, ALPHA * e)
    att = jnp.where(mask, e, NEG_BIG)
    m = jnp.max(att, axis=1, keepdims=True)
    p = jnp.exp2(att - m)
    rs = jax.lax.dot_general(p, ones, (((1,), (1,)), ((), ())),
                             preferred_element_type=jnp.float32)
    hp = jax.lax.dot_general(p, Wh, (((1,), (0,)), ((), ())),
                             preferred_element_type=jnp.float32)
    hp = hp * (1.0 / rs)
    return jnp.where(hp > 0, hp, jnp.exp(jnp.minimum(hp, 0.0)) - 1.0)


def _gat2_kernel(x_hbm, adj_hbm, W1_ref, a1_ref, W2_ref, a2_ref, out_hbm,
                 x_vm, adj_vm, out_vm, sem_x, sem_a, sem_o):
    ones = jnp.ones((1, K), dtype=jnp.float32)
    for i in range(B):
        pltpu.make_async_copy(x_hbm.at[i], x_vm.at[i], sem_x.at[i]).start()
        pltpu.make_async_copy(adj_hbm.at[i], adj_vm.at[i], sem_a.at[i]).start()
    for g in range(0, B, GRP):
        for i in range(g, g + GRP):
            pltpu.make_async_copy(x_hbm.at[i], x_vm.at[i], sem_x.at[i]).wait()
            pltpu.make_async_copy(adj_hbm.at[i], adj_vm.at[i], sem_a.at[i]).wait()
        for i in range(g, g + GRP):
            x = x_vm[i]
            mask = adj_vm[i] > 0
            h1 = _gat_block(x, mask, W1_ref, a1_ref, ones, H)
            out_vm[i] = _gat_block(h1, mask, W2_ref, a2_ref, ones, OUT)
            pltpu.make_async_copy(out_vm.at[i], out_hbm.at[i], sem_o.at[i]).start()
    for i in range(B):
        pltpu.make_async_copy(out_vm.at[i], out_hbm.at[i], sem_o.at[i]).wait()


def kernel(x, adj, W1, a1, W2, a2):
    out = pl.pallas_call(
        _gat2_kernel,
        in_specs=[
            pl.BlockSpec(memory_space=pltpu.MemorySpace.HBM),
            pl.BlockSpec(memory_space=pltpu.MemorySpace.HBM),
            pl.BlockSpec((IN, H), lambda: (0, 0)),
            pl.BlockSpec((1, 2 * H), lambda: (0, 0)),
            pl.BlockSpec((H, OUT), lambda: (0, 0)),
            pl.BlockSpec((1, 2 * OUT), lambda: (0, 0)),
        ],
        out_specs=pl.BlockSpec(memory_space=pltpu.MemorySpace.HBM),
        out_shape=jax.ShapeDtypeStruct((B, K, OUT), jnp.float32),
        scratch_shapes=[
            pltpu.VMEM((B, K, IN), jnp.float32),
            pltpu.VMEM((B, K, K), jnp.float32),
            pltpu.VMEM((B, K, OUT), jnp.float32),
            pltpu.SemaphoreType.DMA((B,)),
            pltpu.SemaphoreType.DMA((B,)),
            pltpu.SemaphoreType.DMA((B,)),
        ],
    )(x, adj, W1, a1.reshape(1, 2 * H), W2, a2.reshape(1, 2 * OUT))
    return out


# speedup vs baseline: 1.3395x; 1.0754x over previous
import jax
import jax.numpy as jnp
from jax.experimental import pallas as pl
from jax.experimental.pallas import tpu as pltpu

B, K, IN, H, OUT = 8, 256, 128, 64, 128
BPG = 4
ALPHA = 0.2
NEG_BIG = -9000000000000000.0
LOG2E = 1.4426950408889634


def _gat_block(h, mask, W_ref, a_ref, ones, nh):
    Wh = jax.lax.dot_general(h, W_ref[...], (((1,), (0,)), ((), ())),
                             preferred_element_type=jnp.float32)
    a_s = a_ref[:, :nh] * LOG2E
    a_d = a_ref[:, nh:] * LOG2E
    s = jax.lax.dot_general(Wh, a_s, (((1,), (1,)), ((), ())),
                            preferred_element_type=jnp.float32)
    d = jax.lax.dot_general(a_d, Wh, (((1,), (1,)), ((), ())),
                            preferred_element_type=jnp.float32)
    e = s + d
    e = jnp.maximum(e, ALPHA * e)
    att = jnp.where(mask, e, NEG_BIG)
    m = jnp.max(att, axis=1, keepdims=True)
    p = jnp.exp2(att - m)
    rs = jax.lax.dot_general(p, ones, (((1,), (1,)), ((), ())),
                             preferred_element_type=jnp.float32)
    hp = jax.lax.dot_general(p, Wh, (((1,), (0,)), ((), ())),
                             preferred_element_type=jnp.float32)
    hp = hp * (1.0 / rs)
    return jnp.where(hp > 0, hp, jnp.exp(jnp.minimum(hp, 0.0)) - 1.0)


def _gat2_kernel(x_ref, adj_ref, W1_ref, a1_ref, W2_ref, a2_ref, out_ref):
    ones = jnp.ones((1, K), dtype=jnp.float32)
    for i in range(BPG):
        x = x_ref[i]
        mask = adj_ref[i] > 0
        h1 = _gat_block(x, mask, W1_ref, a1_ref, ones, H)
        out_ref[i] = _gat_block(h1, mask, W2_ref, a2_ref, ones, OUT)


def kernel(x, adj, W1, a1, W2, a2):
    grid = (B // BPG,)
    out = pl.pallas_call(
        _gat2_kernel,
        grid=grid,
        in_specs=[
            pl.BlockSpec((BPG, K, IN), lambda b: (b, 0, 0)),
            pl.BlockSpec((BPG, K, K), lambda b: (b, 0, 0)),
            pl.BlockSpec((IN, H), lambda b: (0, 0)),
            pl.BlockSpec((1, 2 * H), lambda b: (0, 0)),
            pl.BlockSpec((H, OUT), lambda b: (0, 0)),
            pl.BlockSpec((1, 2 * OUT), lambda b: (0, 0)),
        ],
        out_specs=pl.BlockSpec((BPG, K, OUT), lambda b: (b, 0, 0)),
        out_shape=jax.ShapeDtypeStruct((B, K, OUT), jnp.float32),
        compiler_params=pltpu.CompilerParams(
            dimension_semantics=("parallel",)),
    )(x, adj, W1, a1.reshape(1, 2 * H), W2, a2.reshape(1, 2 * OUT))
    return out
